# Initial kernel scaffold; baseline (speedup 1.0000x reference)
import jax
import jax.numpy as jnp
from jax.experimental import pallas as pl


def kernel(mu_upper, edge_index, W, a):
    raise NotImplementedError("write your pallas kernel here")


# speedup vs baseline: 7.7422x; 7.7422x over previous
import functools

import jax
import jax.numpy as jnp
from jax import lax
from jax.experimental import pallas as pl
from jax.experimental.pallas import tpu as pltpu
from jax.experimental.pallas import tpu_sc as plsc

N_NODES = 10000
N_EDGES = 160000
F = 128
FH = 64
N_STEPS = 5
LR = 0.1
SLOPE = 0.2

NC = 2
NS = 16
EPT = N_EDGES // NS
ROWS = (EPT + 127) // 128
EPAD = ROWS * 128
NPT = 640
WZR = 128


def _tc_transform_body(mu_ref, w_ref, out_ref):
    out_ref[0] = lax.dot_general(
        mu_ref[...], w_ref[...], (((1,), (1,)), ((), ())),
        preferred_element_type=jnp.float32)


def _tc_transform(mu_upper, W):
    RB = 2000
    return pl.pallas_call(
        _tc_transform_body,
        grid=(NC, N_NODES // RB),
        in_specs=[pl.BlockSpec((RB, F), lambda c, i: (i, 0)),
                  pl.BlockSpec((FH, F), lambda c, i: (c, 0))],
        out_specs=pl.BlockSpec((1, RB, FH), lambda c, i: (c, i, 0)),
        out_shape=jax.ShapeDtypeStruct((NC, N_NODES, FH), jnp.float32),
    )(mu_upper, W)


def _tc_node_body(agg_a_ref, agg_b_ref, mu_ref, amat_ref, mu_out, err_out, s_out):
    muhat = jnp.concatenate(
        [jax.nn.relu(agg_a_ref[...]), jax.nn.relu(agg_b_ref[...])], axis=1)
    err = mu_ref[...] - muhat
    err_out[...] = err
    mu_out[...] = mu_ref[...] - LR * err
    s_out[...] = lax.dot_general(
        err, amat_ref[...], (((1,), (1,)), ((), ())),
        preferred_element_type=jnp.float32)


def _tc_node(agg_a, agg_b, mu, amat):
    RB = 2000
    return pl.pallas_call(
        _tc_node_body,
        grid=(N_NODES // RB,),
        in_specs=[pl.BlockSpec((RB, FH), lambda i: (i, 0)),
                  pl.BlockSpec((RB, FH), lambda i: (i, 0)),
                  pl.BlockSpec((RB, F), lambda i: (i, 0)),
                  pl.BlockSpec((2, F), lambda i: (0, 0))],
        out_specs=[pl.BlockSpec((RB, F), lambda i: (i, 0)),
                   pl.BlockSpec((RB, F), lambda i: (i, 0)),
                   pl.BlockSpec((RB, 2), lambda i: (i, 0))],
        out_shape=[jax.ShapeDtypeStruct((N_NODES, F), jnp.float32),
                   jax.ShapeDtypeStruct((N_NODES, F), jnp.float32),
                   jax.ShapeDtypeStruct((N_NODES, 2), jnp.float32)],
    )(agg_a, agg_b, mu, amat)


@functools.partial(
    pl.kernel,
    out_type=[jax.ShapeDtypeStruct((NS, ROWS, 128), jnp.float32),
              jax.ShapeDtypeStruct((NC, N_NODES, FH), jnp.float32)],
    mesh=plsc.VectorSubcoreMesh(core_axis_name="c", subcore_axis_name="s"),
    compiler_params=pltpu.CompilerParams(needs_layout_passes=False,
                                         use_tc_tiling_on_sc=False),
    scratch_types=[
        pltpu.VMEM((ROWS, 128), jnp.int32),
        pltpu.VMEM((ROWS, 128), jnp.int32),
        pltpu.VMEM((ROWS, 128), jnp.float32),
        pltpu.VMEM((128,), jnp.float32),
        pltpu.VMEM((128,), jnp.float32),
        pltpu.VMEM((128, FH), jnp.float32),
        pltpu.VMEM((WZR, FH), jnp.float32),
        pltpu.VMEM_SHARED((N_NODES, FH), jnp.float32),
        pltpu.VMEM_SHARED((N_NODES,), jnp.float32),
        pltpu.VMEM_SHARED((N_NODES,), jnp.float32),
        pltpu.VMEM_SHARED((N_NODES,), jnp.float32),
        pltpu.SemaphoreType.DMA,
    ],
)
def _sc_edge(src_hbm, dst_hbm, s1_hbm, s2_hbm, t2_hbm,
             alpha_out, agg_out,
             src_t, dst_t, ex_t, g1, g2, rows0, wz,
             shared_agg, shared_se, shared_s1, shared_s2, gsem):
    cid = lax.axis_index("c")
    sid = lax.axis_index("s")
    zero16 = jnp.zeros((16,), jnp.float32)

    def node_hops(fn):
        @pl.when(sid < NS - 1)
        def _():
            for h in range(NPT // WZR):
                fn(pl.multiple_of(sid * NPT + h * WZR, WZR), WZR)

        @pl.when(sid == NS - 1)
        def _():
            base = (NS - 1) * NPT
            for h in range((N_NODES - base) // WZR):
                fn(base + h * WZR, WZR)
            tail = (N_NODES - base) % WZR
            if tail:
                fn(N_NODES - tail, tail)

    def z_wz(i, c):
        r = i // (FH // 16)
        k = (i % (FH // 16)) * 16
        wz[r, pl.ds(k, 16)] = zero16
        return c
    lax.fori_loop(0, WZR * (FH // 16), z_wz, 0)

    pltpu.sync_copy(src_hbm.at[sid], src_t)
    pltpu.sync_copy(dst_hbm.at[sid], dst_t)

    def z_g(i, c):
        g1[pl.ds(i * 16, 16)] = zero16
        g2[pl.ds(i * 16, 16)] = zero16
        return c
    lax.fori_loop(0, 8, z_g, 0)
    node_hops(lambda off, n: pltpu.sync_copy(
        wz.at[pl.ds(0, n), :], shared_agg.at[pl.ds(off, n), :]))
    node_hops(lambda off, n: pltpu.sync_copy(
        g1.at[pl.ds(0, n)], shared_se.at[pl.ds(off, n)]))

    def stage_s(off, n):
        pltpu.sync_copy(s1_hbm.at[pl.ds(off, n)], g1.at[pl.ds(0, n)])
        pltpu.sync_copy(g1.at[pl.ds(0, n)], shared_s1.at[pl.ds(off, n)])
        pltpu.sync_copy(s2_hbm.at[pl.ds(off, n)], g2.at[pl.ds(0, n)])
        pltpu.sync_copy(g2.at[pl.ds(0, n)], shared_s2.at[pl.ds(off, n)])
    node_hops(stage_s)

    plsc.subcore_barrier()

    def p1(j, c):
        pltpu.sync_copy(shared_s1.at[src_t.at[j]], g1)
        pltpu.sync_copy(shared_s2.at[dst_t.at[j]], g2)
        for k in range(8):
            sl = pl.ds(k * 16, 16)
            e = g1[sl] + g2[sl]
            e = jnp.where(e > 0.0, e, SLOPE * e)
            ex_t[j, sl] = jnp.exp(e)
        return c
    lax.fori_loop(0, ROWS, p1, 0)
    for k in range((EPT - (ROWS - 1) * 128) // 16, 8):
        ex_t[ROWS - 1, pl.ds(k * 16, 16)] = zero16

    def p1s(j, c):
        pltpu.async_copy(ex_t.at[j], shared_se.at[dst_t.at[j]], gsem, add=True)
        return c
    lax.fori_loop(0, ROWS, p1s, 0)

    def p1w(j, c):
        pltpu.make_async_copy(ex_t.at[j], shared_se.at[dst_t.at[j]], gsem).wait()
        return c
    lax.fori_loop(0, ROWS, p1w, 0)
    plsc.subcore_barrier()

    def p2(j, c):
        pltpu.sync_copy(shared_se.at[dst_t.at[j]], g2)
        pltpu.sync_copy(t2_hbm.at[cid].at[src_t.at[j]], rows0)
        for k in range(8):
            sl = pl.ds(k * 16, 16)
            seg = g2[sl]
            ex16 = ex_t[j, sl]
            ex_t[j, sl] = jnp.where(seg > 0.0, ex16 / seg, 0.0)

        def scale16(g, c2):
            al16 = ex_t[j, pl.ds(g * 16, 16)]
            for i in range(16):
                av = al16[i]
                e = g * 16 + i
                for f in range(FH // 16):
                    sl = pl.ds(f * 16, 16)
                    rows0[e, sl] = rows0[e, sl] * av
            return c2
        lax.fori_loop(0, 8, scale16, 0)
        pltpu.sync_copy(rows0, shared_agg.at[dst_t.at[j]], add=True)
        return c
    lax.fori_loop(0, ROWS, p2, 0)

    @pl.when(cid == 0)
    def _():
        pltpu.sync_copy(ex_t, alpha_out.at[sid])
    plsc.subcore_barrier()

    def writeout(off, n):
        pltpu.sync_copy(shared_agg.at[pl.ds(off, n), :], wz.at[pl.ds(0, n), :])
        pltpu.sync_copy(wz.at[pl.ds(0, n), :], agg_out.at[cid, pl.ds(off, n), :])
    node_hops(writeout)


def kernel(mu_upper, edge_index, W, a):
    src = edge_index[0]
    dst = edge_index[1]
    pad = jnp.zeros((NS, EPAD - EPT), jnp.int32)
    src_p = jnp.concatenate([src.reshape(NS, EPT), pad], axis=1).reshape(NS, ROWS, 128)
    dst_p = jnp.concatenate([dst.reshape(NS, EPT), pad], axis=1).reshape(NS, ROWS, 128)
    amat = a.reshape(2, F)

    t2 = _tc_transform(mu_upper, W)
    mu = jnp.zeros((N_NODES, F), jnp.float32)
    s1 = jnp.zeros((N_NODES,), jnp.float32)
    s2 = jnp.zeros((N_NODES,), jnp.float32)
    alpha_p = None
    for _ in range(N_STEPS):
        alpha_p, agg = _sc_edge(src_p, dst_p, s1, s2, t2)
        mu, errors, s12 = _tc_node(agg[0], agg[1], mu, amat)
        s1 = s12[:, 0]
        s2 = s12[:, 1]
    alpha = alpha_p.reshape(NS, EPAD)[:, :EPT].reshape(N_EDGES)
    return mu, errors, alpha
